# Initial kernel scaffold; baseline (speedup 1.0000x reference)
import jax
import jax.numpy as jnp
from jax.experimental import pallas as pl


def kernel(noised_data, t, edge_index, W1, b1, W2, b2, Wm1, bm1, Wm2, bm2):
    raise NotImplementedError("write your pallas kernel here")


# speedup vs baseline: 6.5372x; 6.5372x over previous
import functools
import math

import jax
import jax.numpy as jnp
from jax import lax
from jax.experimental import pallas as pl
from jax.experimental.pallas import tpu as pltpu
from jax.experimental.pallas import tpu_sc as plsc

N = 10000
E = 160000
IN_DIM = 256
T_DIM = 128
HID = 512

NPAD = 10240
CH = 128
NCHUNK = E // CH
ROWS_PER_TILE = NPAD // 16
MBLK = 1000
GRID = N // MBLK

_f32 = jnp.float32
_i32 = jnp.int32


def _zero_vmem_1d(ref, nwords):
    def body(i, _):
        ref[pl.ds(i * 16, 16)] = jnp.zeros((16,), _f32)
        return 0
    lax.fori_loop(0, nwords // 16, body, 0)


def _zero_vmem_2d(ref, rows, cols):
    per_row = cols // 16
    def body(i, _):
        r = i // per_row
        c = (i % per_row) * 16
        ref[r, pl.ds(c, 16)] = jnp.zeros((16,), _f32)
        return 0
    lax.fori_loop(0, rows * per_row, body, 0)


_sc_mesh = plsc.VectorSubcoreMesh(core_axis_name="c", subcore_axis_name="s")


@functools.partial(
    pl.kernel,
    mesh=_sc_mesh,
    out_type=jax.ShapeDtypeStruct((2, NPAD), _f32),
    scratch_types=[
        pltpu.VMEM((CH,), _i32),
        pltpu.VMEM((CH,), _f32),
        pltpu.VMEM((ROWS_PER_TILE,), _f32),
        pltpu.VMEM_SHARED((NPAD,), _f32),
    ],
)
def _deg_call(dst_hbm, out_hbm, dst_v, ones_v, zvec_v, acc_sh):
    c = lax.axis_index("c")
    s = lax.axis_index("s")
    _zero_vmem_1d(zvec_v, ROWS_PER_TILE)
    _zero_vmem_1d(ones_v, CH)
    def fill1(i, _):
        ones_v[pl.ds(i * 16, 16)] = jnp.ones((16,), _f32)
        return 0
    lax.fori_loop(0, CH // 16, fill1, 0)
    pltpu.sync_copy(zvec_v, acc_sh.at[pl.ds(s * ROWS_PER_TILE, ROWS_PER_TILE)])
    plsc.subcore_barrier()

    nch = NCHUNK // 2
    n = jnp.where(s == 0, nch // 16 + 1, nch // 16)

    def body(r, _):
        cid = c * nch + r * 16 + s
        base = pl.multiple_of(cid * CH, 8)
        pltpu.sync_copy(dst_hbm.at[pl.ds(base, CH)], dst_v)
        pltpu.sync_copy(ones_v, acc_sh.at[dst_v], add=True)
        return 0

    lax.fori_loop(0, n, body, 0)
    plsc.subcore_barrier()
    sl = pl.ds(s * ROWS_PER_TILE, ROWS_PER_TILE)
    pltpu.sync_copy(acc_sh.at[sl], out_hbm.at[c, sl])


@functools.partial(
    pl.kernel,
    mesh=_sc_mesh,
    out_type=(
        jax.ShapeDtypeStruct((2, NPAD, 128), _f32),
        jax.ShapeDtypeStruct((2, NPAD), _f32),
    ),
    scratch_types=[
        pltpu.VMEM((CH,), _i32),
        pltpu.VMEM((CH,), _i32),
        pltpu.VMEM((CH, 128), _f32),
        pltpu.VMEM((CH,), _f32),
        pltpu.VMEM((CH, 128), _f32),
        pltpu.VMEM((ROWS_PER_TILE,), _f32),
        pltpu.VMEM_SHARED((NPAD, 128), _f32),
        pltpu.VMEM_SHARED((NPAD,), _f32),
        pltpu.SemaphoreType.DMA,
    ],
)
def _prop1_call(p0_hbm, dinv_hbm, src_hbm, dst_hbm, s1_hbm, sv_hbm,
                src_v, dst_v, rows_v, dval_v, zbuf_v, zvec_v,
                acc_sh, sv_sh, sem):
    c = lax.axis_index("c")
    s = lax.axis_index("s")
    _zero_vmem_2d(zbuf_v, CH, 128)
    _zero_vmem_1d(zvec_v, ROWS_PER_TILE)
    r0 = s * ROWS_PER_TILE
    for j in range(ROWS_PER_TILE // CH):
        pltpu.sync_copy(zbuf_v, acc_sh.at[pl.ds(r0 + j * CH, CH), :])
    pltpu.sync_copy(zvec_v, sv_sh.at[pl.ds(r0, ROWS_PER_TILE)])
    plsc.subcore_barrier()

    n = jnp.where(s < 2, NCHUNK // 16 + 1, NCHUNK // 16)
    off = c * N

    def body(r, _):
        cid = r * 16 + s
        base = pl.multiple_of(cid * CH, 8)
        pltpu.sync_copy(src_hbm.at[pl.ds(base, CH)], src_v)
        pltpu.sync_copy(dst_hbm.at[pl.ds(base, CH)], dst_v)

        @pl.when((cid & 1) == c)
        def _():
            pltpu.async_copy(dinv_hbm.at[src_v], dval_v, sem).wait()
            pltpu.sync_copy(dval_v, sv_sh.at[dst_v], add=True)

        for k in range(CH // 16):
            src_v[pl.ds(k * 16, 16)] = src_v[pl.ds(k * 16, 16)] + off
        pltpu.async_copy(p0_hbm.at[src_v], rows_v, sem).wait()
        pltpu.sync_copy(rows_v, acc_sh.at[dst_v], add=True)
        return 0

    lax.fori_loop(0, n, body, 0)
    plsc.subcore_barrier()
    sl = pl.ds(r0, ROWS_PER_TILE)
    pltpu.sync_copy(acc_sh.at[sl, :], s1_hbm.at[c, sl, :])
    pltpu.sync_copy(sv_sh.at[sl], sv_hbm.at[c, sl])


@functools.partial(
    pl.kernel,
    mesh=_sc_mesh,
    out_type=jax.ShapeDtypeStruct((4, NPAD, 128), _f32),
    scratch_types=[
        pltpu.VMEM((CH,), _i32),
        pltpu.VMEM((CH,), _i32),
        pltpu.VMEM((CH, 128), _f32),
        pltpu.VMEM((CH, 128), _f32),
        pltpu.VMEM_SHARED((NPAD, 128), _f32),
        pltpu.SemaphoreType.DMA,
    ],
)
def _prop2_call(t_hbm, src_hbm, dst_hbm, s2_hbm,
                src_v, dst_v, rows_v, zbuf_v, acc_sh, sem):
    c = lax.axis_index("c")
    s = lax.axis_index("s")
    _zero_vmem_2d(zbuf_v, CH, 128)
    r0 = s * ROWS_PER_TILE
    n = jnp.where(s < 2, NCHUNK // 16 + 1, NCHUNK // 16)

    for j in range(2):
        b = c * 2 + j
        off = b * N
        for q in range(ROWS_PER_TILE // CH):
            pltpu.sync_copy(zbuf_v, acc_sh.at[pl.ds(r0 + q * CH, CH), :])
        plsc.subcore_barrier()

        def body(r, _):
            cid = r * 16 + s
            base = pl.multiple_of(cid * CH, 8)
            pltpu.sync_copy(src_hbm.at[pl.ds(base, CH)], src_v)
            pltpu.sync_copy(dst_hbm.at[pl.ds(base, CH)], dst_v)
            for k in range(CH // 16):
                src_v[pl.ds(k * 16, 16)] = src_v[pl.ds(k * 16, 16)] + off
            pltpu.async_copy(t_hbm.at[src_v], rows_v, sem).wait()
            pltpu.sync_copy(rows_v, acc_sh.at[dst_v], add=True)
            return 0

        lax.fori_loop(0, n, body, 0)
        plsc.subcore_barrier()
        sl = pl.ds(r0, ROWS_PER_TILE)
        pltpu.sync_copy(acc_sh.at[sl, :], s2_hbm.at[b, sl, :])


def _tc1_body(deg2_ref, x_ref, dinv_ref, p0_ref):
    deg = deg2_ref[0] + deg2_ref[1]
    dinv = lax.rsqrt(jnp.clip(deg, 1.0, None))
    dinv_ref[...] = dinv
    p0 = x_ref[...] * dinv
    p0_ref[0] = p0[:, :128]
    p0_ref[1] = p0[:, 128:]


def _tc1_call(deg2, x):
    return pl.pallas_call(
        _tc1_body,
        grid=(GRID,),
        in_specs=[
            pl.BlockSpec((2, MBLK, 1), lambda i: (0, i, 0)),
            pl.BlockSpec((MBLK, IN_DIM), lambda i: (i, 0)),
        ],
        out_specs=[
            pl.BlockSpec((MBLK, 1), lambda i: (i, 0)),
            pl.BlockSpec((2, MBLK, 128), lambda i: (0, i, 0)),
        ],
        out_shape=[
            jax.ShapeDtypeStruct((N, 1), _f32),
            jax.ShapeDtypeStruct((2, N, 128), _f32),
        ],
    )(deg2, x)


def _tc2_body(s1_ref, sv2_ref, dinv_ref, temb_ref, w1a_ref, w1b_ref, b1_ref,
              w2_ref, t_ref, s_ref):
    dinv = dinv_ref[...]
    sv = sv2_ref[0] + sv2_ref[1]
    sg = dinv * sv
    s_ref[...] = sg
    x = jnp.concatenate([s1_ref[0], s1_ref[1]], axis=1) * dinv
    v1b = jnp.dot(temb_ref[...], w1b_ref[...],
                  preferred_element_type=_f32) + b1_ref[...]
    h1 = jnp.dot(x, w1a_ref[...], preferred_element_type=_f32) + sg * v1b
    h1 = jnp.maximum(h1, 0.0)
    tt = jnp.dot(h1 * dinv, w2_ref[...], preferred_element_type=_f32)
    for b in range(4):
        t_ref[b] = tt[:, b * 128:(b + 1) * 128]


def _tc2_call(s1, sv2, dinv, temb, w1a, w1b, b1, w2):
    return pl.pallas_call(
        _tc2_body,
        grid=(GRID,),
        in_specs=[
            pl.BlockSpec((2, MBLK, 128), lambda i: (0, i, 0)),
            pl.BlockSpec((2, MBLK, 1), lambda i: (0, i, 0)),
            pl.BlockSpec((MBLK, 1), lambda i: (i, 0)),
            pl.BlockSpec((1, T_DIM), lambda i: (0, 0)),
            pl.BlockSpec((IN_DIM, HID), lambda i: (0, 0)),
            pl.BlockSpec((T_DIM, HID), lambda i: (0, 0)),
            pl.BlockSpec((1, HID), lambda i: (0, 0)),
            pl.BlockSpec((HID, HID), lambda i: (0, 0)),
        ],
        out_specs=[
            pl.BlockSpec((4, MBLK, 128), lambda i: (0, i, 0)),
            pl.BlockSpec((MBLK, 1), lambda i: (i, 0)),
        ],
        out_shape=[
            jax.ShapeDtypeStruct((4, N, 128), _f32),
            jax.ShapeDtypeStruct((N, 1), _f32),
        ],
    )(s1, sv2, dinv, temb, w1a, w1b, b1, w2)


def _tc3_body(s2_ref, dinv_ref, s_ref, b2_ref, wm1_ref, bm1_ref, wm2_ref,
              bm2_ref, out_ref):
    dinv = dinv_ref[...]
    h2 = jnp.concatenate([s2_ref[0], s2_ref[1], s2_ref[2], s2_ref[3]],
                         axis=1) * dinv + s_ref[...] * b2_ref[...]
    z = jnp.maximum(jnp.dot(h2, wm1_ref[...], preferred_element_type=_f32)
                    + bm1_ref[...], 0.0)
    out_ref[...] = jnp.dot(z, wm2_ref[...],
                           preferred_element_type=_f32) + bm2_ref[...]


def _tc3_call(s2, dinv, s, b2, wm1, bm1, wm2, bm2):
    return pl.pallas_call(
        _tc3_body,
        grid=(GRID,),
        in_specs=[
            pl.BlockSpec((4, MBLK, 128), lambda i: (0, i, 0)),
            pl.BlockSpec((MBLK, 1), lambda i: (i, 0)),
            pl.BlockSpec((MBLK, 1), lambda i: (i, 0)),
            pl.BlockSpec((1, HID), lambda i: (0, 0)),
            pl.BlockSpec((HID, HID), lambda i: (0, 0)),
            pl.BlockSpec((1, HID), lambda i: (0, 0)),
            pl.BlockSpec((HID, IN_DIM), lambda i: (0, 0)),
            pl.BlockSpec((1, IN_DIM), lambda i: (0, 0)),
        ],
        out_specs=pl.BlockSpec((MBLK, IN_DIM), lambda i: (i, 0)),
        out_shape=jax.ShapeDtypeStruct((N, IN_DIM), _f32),
    )(s2, dinv, s, b2, wm1, bm1, wm2, bm2)


def kernel(noised_data, t, edge_index, W1, b1, W2, b2, Wm1, bm1, Wm2, bm2):
    x = noised_data[0]
    src = edge_index[0]
    dst = edge_index[1]

    half = T_DIM // 2
    freq = jnp.exp(jnp.arange(half, dtype=_f32) * (-math.log(10000.0) / (half - 1)))
    ang = t[0] * freq
    temb = jnp.concatenate([jnp.sin(ang), jnp.cos(ang)])[None]

    deg2 = _deg_call(dst)
    dinv, p0 = _tc1_call(deg2.reshape(2, NPAD, 1), x)
    s1, sv2 = _prop1_call(p0.reshape(2 * N, 128),
                          dinv.reshape(N), src, dst)
    tmat, s = _tc2_call(s1, sv2.reshape(2, NPAD, 1), dinv, temb,
                        W1[:IN_DIM], W1[IN_DIM:], b1[None], W2)
    s2 = _prop2_call(tmat.reshape(4 * N, 128), src, dst)
    out = _tc3_call(s2, dinv, s, b2[None], Wm1, bm1[None],
                    Wm2, bm2[None])
    return out[None]


# speedup vs baseline: 13.4383x; 2.0556x over previous
import functools
import math

import jax
import jax.numpy as jnp
from jax import lax
from jax.experimental import pallas as pl
from jax.experimental.pallas import tpu as pltpu
from jax.experimental.pallas import tpu_sc as plsc

N = 10000
E = 160000
IN_DIM = 256
T_DIM = 128
HID = 512

NPAD = 10240
CH = 128
NCHUNK = E // CH
NGRP = 10
CPAD = 1280
ROWS_PER_TILE = NPAD // 16
MBLK = 1000
GRID = N // MBLK

_f32 = jnp.float32
_i32 = jnp.int32


def _zero_vmem_1d(ref, nwords):
    def body(i, _):
        ref[pl.ds(i * 16, 16)] = jnp.zeros((16,), _f32)
        return 0
    lax.fori_loop(0, nwords // 16, body, 0)


def _zero_vmem_2d(ref, rows, cols):
    per_row = cols // 16
    def body(i, _):
        r = i // per_row
        c = (i % per_row) * 16
        ref[r, pl.ds(c, 16)] = jnp.zeros((16,), _f32)
        return 0
    lax.fori_loop(0, rows * per_row, body, 0)


_sc_mesh = plsc.VectorSubcoreMesh(core_axis_name="c", subcore_axis_name="s")


@functools.partial(
    pl.kernel,
    mesh=_sc_mesh,
    out_type=jax.ShapeDtypeStruct((2, NPAD), _f32),
    scratch_types=[
        pltpu.VMEM((CH,), _i32),
        pltpu.VMEM((CH,), _f32),
        pltpu.VMEM((ROWS_PER_TILE,), _f32),
        pltpu.VMEM_SHARED((NPAD,), _f32),
    ],
)
def _deg_call(dst_hbm, out_hbm, dst_v, ones_v, zvec_v, acc_sh):
    c = lax.axis_index("c")
    s = lax.axis_index("s")
    _zero_vmem_1d(zvec_v, ROWS_PER_TILE)
    _zero_vmem_1d(ones_v, CH)
    def fill1(i, _):
        ones_v[pl.ds(i * 16, 16)] = jnp.ones((16,), _f32)
        return 0
    lax.fori_loop(0, CH // 16, fill1, 0)
    pltpu.sync_copy(zvec_v, acc_sh.at[pl.ds(s * ROWS_PER_TILE, ROWS_PER_TILE)])
    plsc.subcore_barrier()

    nch = NCHUNK // 2
    n = jnp.where(s == 0, nch // 16 + 1, nch // 16)

    def body(r, _):
        cid = c * nch + r * 16 + s
        base = pl.multiple_of(cid * CH, 8)
        pltpu.sync_copy(dst_hbm.at[pl.ds(base, CH)], dst_v)
        pltpu.sync_copy(ones_v, acc_sh.at[dst_v], add=True)
        return 0

    lax.fori_loop(0, n, body, 0)
    plsc.subcore_barrier()
    sl = pl.ds(s * ROWS_PER_TILE, ROWS_PER_TILE)
    pltpu.sync_copy(acc_sh.at[sl], out_hbm.at[c, sl])


def _edge_pipeline(tbl_hbm, src2_hbm, dst2_hbm, dinv_hbm, acc_sh, sv_sh,
                   sidx, didx, adjbuf, rows_v, dval_v, semi, semg, semsv,
                   start, n, off, do_sv):
    GSZ = 8

    def idx_issue(slot, g):
        g8 = start + g * GSZ
        pltpu.async_copy(src2_hbm.at[pl.ds(g8, GSZ), :], sidx.at[slot],
                         semi[slot])
        pltpu.async_copy(dst2_hbm.at[pl.ds(g8, GSZ), :], didx.at[slot],
                         semi[slot])

    def idx_wait(slot, g):
        g8 = start + g * GSZ
        pltpu.make_async_copy(src2_hbm.at[pl.ds(g8, GSZ), :], sidx.at[slot],
                              semi[slot]).wait()
        pltpu.make_async_copy(dst2_hbm.at[pl.ds(g8, GSZ), :], didx.at[slot],
                              semi[slot]).wait()

    def fire(slot, u):
        w = u % 2
        if do_sv:
            pltpu.async_copy(dinv_hbm.at[sidx.at[slot, u]], dval_v.at[w],
                             semsv[w])
        for k in range(CH // 16):
            adjbuf[w, pl.ds(k * 16, 16)] = sidx[slot, u, pl.ds(k * 16, 16)] + off
        pltpu.async_copy(tbl_hbm.at[adjbuf.at[w]], rows_v.at[w], semg[w])

    def drain(slot, u):
        w = u % 2
        pltpu.make_async_copy(tbl_hbm.at[adjbuf.at[w]], rows_v.at[w],
                              semg[w]).wait()
        pltpu.sync_copy(rows_v.at[w], acc_sh.at[didx.at[slot, u]], add=True)
        if do_sv:
            pltpu.make_async_copy(dinv_hbm.at[sidx.at[slot, u]],
                                  dval_v.at[w], semsv[w]).wait()
            pltpu.sync_copy(dval_v.at[w], sv_sh.at[didx.at[slot, u]],
                            add=True)

    idx_issue(0, 0)

    def body(gg, _):
        for h in (0, 1):
            g = gg * 2 + h
            idx_wait(h, g)
            for u in range(GSZ):
                q_drain = g * GSZ + u - 2
                slot_d, u_d = ((1 - h, u + 6) if u < 2 else (h, u - 2))

                @pl.when(jnp.logical_and(q_drain >= 0, q_drain < n))
                def _(slot_d=slot_d, u_d=u_d):
                    drain(slot_d, u_d)

                @pl.when(g * GSZ + u < n)
                def _(h=h, u=u):
                    fire(h, u)

                if u == 4:
                    @pl.when(g + 1 < NGRP)
                    def _(h=h, g=g):
                        idx_issue(1 - h, g + 1)
        return 0

    lax.fori_loop(0, NGRP // 2, body, 0)

    hl = (NGRP - 1) % 2
    for e in range(2):
        @pl.when((NGRP - 1) * GSZ + 6 + e < n)
        def _(e=e):
            drain(hl, 6 + e)


@functools.partial(
    pl.kernel,
    mesh=_sc_mesh,
    out_type=(
        jax.ShapeDtypeStruct((2, NPAD, 128), _f32),
        jax.ShapeDtypeStruct((2, NPAD), _f32),
    ),
    scratch_types=[
        pltpu.VMEM((2, 8, CH), _i32),
        pltpu.VMEM((2, 8, CH), _i32),
        pltpu.VMEM((2, CH), _i32),
        pltpu.VMEM((2, CH, 128), _f32),
        pltpu.VMEM((2, CH), _f32),
        pltpu.VMEM((64, 128), _f32),
        pltpu.VMEM((ROWS_PER_TILE,), _f32),
        pltpu.VMEM_SHARED((NPAD, 128), _f32),
        pltpu.VMEM_SHARED((NPAD,), _f32),
        pltpu.SemaphoreType.DMA,
        pltpu.SemaphoreType.DMA,
        pltpu.SemaphoreType.DMA,
        pltpu.SemaphoreType.DMA,
        pltpu.SemaphoreType.DMA,
        pltpu.SemaphoreType.DMA,
    ],
)
def _prop1_call(p0_hbm, dinv_hbm, src2_hbm, dst2_hbm, s1_hbm, sv_hbm,
                sidx, didx, adjbuf, rows_v, dval_v, zbuf_v, zvec_v,
                acc_sh, sv_sh,
                semi0, semi1, semg0, semg1, semsv0, semsv1):
    c = lax.axis_index("c")
    s = lax.axis_index("s")
    _zero_vmem_2d(zbuf_v, 64, 128)
    _zero_vmem_1d(zvec_v, ROWS_PER_TILE)
    r0 = s * ROWS_PER_TILE
    for j in range(ROWS_PER_TILE // 64):
        pltpu.sync_copy(zbuf_v, acc_sh.at[pl.ds(r0 + j * 64, 64), :])
    pltpu.sync_copy(zvec_v, sv_sh.at[pl.ds(r0, ROWS_PER_TILE)])
    plsc.subcore_barrier()

    start = s * 80
    n = jnp.minimum(80, NCHUNK - s * 80)
    _edge_pipeline(p0_hbm, src2_hbm, dst2_hbm, dinv_hbm, acc_sh, sv_sh,
                   sidx, didx, adjbuf, rows_v, dval_v,
                   (semi0, semi1), (semg0, semg1), (semsv0, semsv1),
                   start, n, c * N, do_sv=True)

    plsc.subcore_barrier()
    sl = pl.ds(r0, ROWS_PER_TILE)
    pltpu.sync_copy(acc_sh.at[sl, :], s1_hbm.at[c, sl, :])
    pltpu.sync_copy(sv_sh.at[sl], sv_hbm.at[c, sl])


@functools.partial(
    pl.kernel,
    mesh=_sc_mesh,
    out_type=jax.ShapeDtypeStruct((4, NPAD, 128), _f32),
    scratch_types=[
        pltpu.VMEM((2, 8, CH), _i32),
        pltpu.VMEM((2, 8, CH), _i32),
        pltpu.VMEM((2, CH), _i32),
        pltpu.VMEM((2, CH, 128), _f32),
        pltpu.VMEM((64, 128), _f32),
        pltpu.VMEM_SHARED((NPAD, 128), _f32),
        pltpu.SemaphoreType.DMA,
        pltpu.SemaphoreType.DMA,
        pltpu.SemaphoreType.DMA,
        pltpu.SemaphoreType.DMA,
    ],
)
def _prop2_call(t_hbm, src2_hbm, dst2_hbm, s2_hbm,
                sidx, didx, adjbuf, rows_v, zbuf_v, acc_sh,
                semi0, semi1, semg0, semg1):
    c = lax.axis_index("c")
    s = lax.axis_index("s")
    _zero_vmem_2d(zbuf_v, 64, 128)
    r0 = s * ROWS_PER_TILE
    start = s * 80
    n = jnp.minimum(80, NCHUNK - s * 80)

    for j in range(2):
        b = c * 2 + j
        for q in range(ROWS_PER_TILE // 64):
            pltpu.sync_copy(zbuf_v, acc_sh.at[pl.ds(r0 + q * 64, 64), :])
        plsc.subcore_barrier()

        _edge_pipeline(t_hbm, src2_hbm, dst2_hbm, None, acc_sh, None,
                       sidx, didx, adjbuf, rows_v, None,
                       (semi0, semi1), (semg0, semg1), None,
                       start, n, b * N, do_sv=False)

        plsc.subcore_barrier()
        sl = pl.ds(r0, ROWS_PER_TILE)
        pltpu.sync_copy(acc_sh.at[sl, :], s2_hbm.at[b, sl, :])


def _tc1_body(deg2_ref, x_ref, dinv_ref, p0_ref):
    deg = deg2_ref[0] + deg2_ref[1]
    dinv = lax.rsqrt(jnp.clip(deg, 1.0, None))
    dinv_ref[...] = dinv
    p0 = x_ref[...] * dinv
    p0_ref[0] = p0[:, :128]
    p0_ref[1] = p0[:, 128:]


def _tc1_call(deg2, x):
    return pl.pallas_call(
        _tc1_body,
        grid=(GRID,),
        in_specs=[
            pl.BlockSpec((2, MBLK, 1), lambda i: (0, i, 0)),
            pl.BlockSpec((MBLK, IN_DIM), lambda i: (i, 0)),
        ],
        out_specs=[
            pl.BlockSpec((MBLK, 1), lambda i: (i, 0)),
            pl.BlockSpec((2, MBLK, 128), lambda i: (0, i, 0)),
        ],
        out_shape=[
            jax.ShapeDtypeStruct((N, 1), _f32),
            jax.ShapeDtypeStruct((2, N, 128), _f32),
        ],
    )(deg2, x)


def _tc2_body(s1_ref, sv2_ref, dinv_ref, temb_ref, w1a_ref, w1b_ref, b1_ref,
              w2_ref, t_ref, s_ref):
    dinv = dinv_ref[...]
    sv = (sv2_ref[0] + sv2_ref[1]) * 0.5
    sg = dinv * sv
    s_ref[...] = sg
    x = jnp.concatenate([s1_ref[0], s1_ref[1]], axis=1) * dinv
    v1b = jnp.dot(temb_ref[...], w1b_ref[...],
                  preferred_element_type=_f32) + b1_ref[...]
    h1 = jnp.dot(x, w1a_ref[...], preferred_element_type=_f32) + sg * v1b
    h1 = jnp.maximum(h1, 0.0)
    tt = jnp.dot(h1 * dinv, w2_ref[...], preferred_element_type=_f32)
    for b in range(4):
        t_ref[b] = tt[:, b * 128:(b + 1) * 128]


def _tc2_call(s1, sv2, dinv, temb, w1a, w1b, b1, w2):
    return pl.pallas_call(
        _tc2_body,
        grid=(GRID,),
        in_specs=[
            pl.BlockSpec((2, MBLK, 128), lambda i: (0, i, 0)),
            pl.BlockSpec((2, MBLK, 1), lambda i: (0, i, 0)),
            pl.BlockSpec((MBLK, 1), lambda i: (i, 0)),
            pl.BlockSpec((1, T_DIM), lambda i: (0, 0)),
            pl.BlockSpec((IN_DIM, HID), lambda i: (0, 0)),
            pl.BlockSpec((T_DIM, HID), lambda i: (0, 0)),
            pl.BlockSpec((1, HID), lambda i: (0, 0)),
            pl.BlockSpec((HID, HID), lambda i: (0, 0)),
        ],
        out_specs=[
            pl.BlockSpec((4, MBLK, 128), lambda i: (0, i, 0)),
            pl.BlockSpec((MBLK, 1), lambda i: (i, 0)),
        ],
        out_shape=[
            jax.ShapeDtypeStruct((4, N, 128), _f32),
            jax.ShapeDtypeStruct((N, 1), _f32),
        ],
    )(s1, sv2, dinv, temb, w1a, w1b, b1, w2)


def _tc3_body(s2_ref, dinv_ref, s_ref, b2_ref, wm1_ref, bm1_ref, wm2_ref,
              bm2_ref, out_ref):
    dinv = dinv_ref[...]
    h2 = jnp.concatenate([s2_ref[0], s2_ref[1], s2_ref[2], s2_ref[3]],
                         axis=1) * dinv + s_ref[...] * b2_ref[...]
    z = jnp.maximum(jnp.dot(h2, wm1_ref[...], preferred_element_type=_f32)
                    + bm1_ref[...], 0.0)
    out_ref[...] = jnp.dot(z, wm2_ref[...],
                           preferred_element_type=_f32) + bm2_ref[...]


def _tc3_call(s2, dinv, s, b2, wm1, bm1, wm2, bm2):
    return pl.pallas_call(
        _tc3_body,
        grid=(GRID,),
        in_specs=[
            pl.BlockSpec((4, MBLK, 128), lambda i: (0, i, 0)),
            pl.BlockSpec((MBLK, 1), lambda i: (i, 0)),
            pl.BlockSpec((MBLK, 1), lambda i: (i, 0)),
            pl.BlockSpec((1, HID), lambda i: (0, 0)),
            pl.BlockSpec((HID, HID), lambda i: (0, 0)),
            pl.BlockSpec((1, HID), lambda i: (0, 0)),
            pl.BlockSpec((HID, IN_DIM), lambda i: (0, 0)),
            pl.BlockSpec((1, IN_DIM), lambda i: (0, 0)),
        ],
        out_specs=pl.BlockSpec((MBLK, IN_DIM), lambda i: (i, 0)),
        out_shape=jax.ShapeDtypeStruct((N, IN_DIM), _f32),
    )(s2, dinv, s, b2, wm1, bm1, wm2, bm2)


def kernel(noised_data, t, edge_index, W1, b1, W2, b2, Wm1, bm1, Wm2, bm2):
    x = noised_data[0]
    dst = edge_index[1]
    ei_pad = jnp.pad(edge_index, ((0, 0), (0, CPAD * CH - E)))
    src2 = ei_pad[0].reshape(CPAD, CH)
    dst2 = ei_pad[1].reshape(CPAD, CH)

    half = T_DIM // 2
    freq = jnp.exp(jnp.arange(half, dtype=_f32) * (-math.log(10000.0) / (half - 1)))
    ang = t[0] * freq
    temb = jnp.concatenate([jnp.sin(ang), jnp.cos(ang)])[None]

    deg2 = _deg_call(dst)
    dinv, p0 = _tc1_call(deg2.reshape(2, NPAD, 1), x)
    s1, sv2 = _prop1_call(p0.reshape(2 * N, 128),
                          dinv.reshape(N), src2, dst2)
    tmat, s = _tc2_call(s1, sv2.reshape(2, NPAD, 1), dinv, temb,
                        W1[:IN_DIM], W1[IN_DIM:], b1[None], W2)
    s2 = _prop2_call(tmat.reshape(4 * N, 128), src2, dst2)
    out = _tc3_call(s2, dinv, s, b2[None], Wm1, bm1[None],
                    Wm2, bm2[None])
    return out[None]


# speedup vs baseline: 13.4436x; 1.0004x over previous
import functools
import math

import jax
import jax.numpy as jnp
from jax import lax
from jax.experimental import pallas as pl
from jax.experimental.pallas import tpu as pltpu
from jax.experimental.pallas import tpu_sc as plsc

N = 10000
E = 160000
IN_DIM = 256
T_DIM = 128
HID = 512

NPAD = 10240
CH = 128
NCHUNK = E // CH
NGRP = 10
CPAD = 1280
ROWS_PER_TILE = NPAD // 16
MBLK = 1000
GRID = N // MBLK

_f32 = jnp.float32
_i32 = jnp.int32


def _zero_vmem_1d(ref, nwords):
    def body(i, _):
        ref[pl.ds(i * 16, 16)] = jnp.zeros((16,), _f32)
        return 0
    lax.fori_loop(0, nwords // 16, body, 0)


def _zero_vmem_2d(ref, rows, cols):
    per_row = cols // 16
    def body(i, _):
        r = i // per_row
        c = (i % per_row) * 16
        ref[r, pl.ds(c, 16)] = jnp.zeros((16,), _f32)
        return 0
    lax.fori_loop(0, rows * per_row, body, 0)


_sc_mesh = plsc.VectorSubcoreMesh(core_axis_name="c", subcore_axis_name="s")


@functools.partial(
    pl.kernel,
    mesh=_sc_mesh,
    out_type=jax.ShapeDtypeStruct((2, NPAD), _f32),
    scratch_types=[
        pltpu.VMEM((CH,), _i32),
        pltpu.VMEM((CH,), _f32),
        pltpu.VMEM((ROWS_PER_TILE,), _f32),
        pltpu.VMEM_SHARED((NPAD,), _f32),
    ],
)
def _deg_call(dst_hbm, out_hbm, dst_v, ones_v, zvec_v, acc_sh):
    c = lax.axis_index("c")
    s = lax.axis_index("s")
    _zero_vmem_1d(zvec_v, ROWS_PER_TILE)
    _zero_vmem_1d(ones_v, CH)
    def fill1(i, _):
        ones_v[pl.ds(i * 16, 16)] = jnp.ones((16,), _f32)
        return 0
    lax.fori_loop(0, CH // 16, fill1, 0)
    pltpu.sync_copy(zvec_v, acc_sh.at[pl.ds(s * ROWS_PER_TILE, ROWS_PER_TILE)])
    plsc.subcore_barrier()

    nch = NCHUNK // 2
    n = jnp.where(s == 0, nch // 16 + 1, nch // 16)

    def body(r, _):
        cid = c * nch + r * 16 + s
        base = pl.multiple_of(cid * CH, 8)
        pltpu.sync_copy(dst_hbm.at[pl.ds(base, CH)], dst_v)
        pltpu.sync_copy(ones_v, acc_sh.at[dst_v], add=True)
        return 0

    lax.fori_loop(0, n, body, 0)
    plsc.subcore_barrier()
    sl = pl.ds(s * ROWS_PER_TILE, ROWS_PER_TILE)
    pltpu.sync_copy(acc_sh.at[sl], out_hbm.at[c, sl])


def _edge_pipeline(tbl_hbm, src2_hbm, dst2_hbm, dinv_hbm, acc_sh, sv_sh,
                   sidx, didx, adjbuf, rows_v, dval_v, semi, semg, semsv,
                   semsc, start, n, off, do_sv):
    GSZ = 8

    def idx_issue(slot, g):
        g8 = start + g * GSZ
        pltpu.async_copy(src2_hbm.at[pl.ds(g8, GSZ), :], sidx.at[slot],
                         semi[slot])
        pltpu.async_copy(dst2_hbm.at[pl.ds(g8, GSZ), :], didx.at[slot],
                         semi[slot])

    def idx_wait(slot, g):
        g8 = start + g * GSZ
        pltpu.make_async_copy(src2_hbm.at[pl.ds(g8, GSZ), :], sidx.at[slot],
                              semi[slot]).wait()
        pltpu.make_async_copy(dst2_hbm.at[pl.ds(g8, GSZ), :], didx.at[slot],
                              semi[slot]).wait()

    def scat_wait(w, slot, u):
        pltpu.make_async_copy(rows_v.at[w], acc_sh.at[didx.at[slot, u]],
                              semsc[w]).wait()

    def fire(slot, u, first_group):
        w = u % 2
        if u >= 2:
            scat_wait(w, slot, u - 2)
        else:
            @pl.when(jnp.logical_not(first_group))
            def _():
                scat_wait(w, 1 - slot, u + 6)
        if do_sv:
            pltpu.async_copy(dinv_hbm.at[sidx.at[slot, u]], dval_v.at[w],
                             semsv[w])
        for k in range(CH // 16):
            adjbuf[w, pl.ds(k * 16, 16)] = sidx[slot, u, pl.ds(k * 16, 16)] + off
        pltpu.async_copy(tbl_hbm.at[adjbuf.at[w]], rows_v.at[w], semg[w])

    def drain(slot, u):
        w = u % 2
        pltpu.make_async_copy(tbl_hbm.at[adjbuf.at[w]], rows_v.at[w],
                              semg[w]).wait()
        pltpu.async_copy(rows_v.at[w], acc_sh.at[didx.at[slot, u]], semsc[w],
                         add=True)
        if do_sv:
            pltpu.make_async_copy(dinv_hbm.at[sidx.at[slot, u]],
                                  dval_v.at[w], semsv[w]).wait()
            pltpu.sync_copy(dval_v.at[w], sv_sh.at[didx.at[slot, u]],
                            add=True)

    idx_issue(0, 0)

    def body(gg, _):
        for h in (0, 1):
            g = gg * 2 + h
            idx_wait(h, g)
            for u in range(GSZ):
                q_drain = g * GSZ + u - 2
                slot_d, u_d = ((1 - h, u + 6) if u < 2 else (h, u - 2))

                @pl.when(jnp.logical_and(q_drain >= 0, q_drain < n))
                def _(slot_d=slot_d, u_d=u_d):
                    drain(slot_d, u_d)

                @pl.when(g * GSZ + u < n)
                def _(h=h, u=u, gg_=gg):
                    fire(h, u, jnp.logical_and(gg_ == 0, h == 0))

                if u == 4:
                    @pl.when(g + 1 < NGRP)
                    def _(h=h, g=g):
                        idx_issue(1 - h, g + 1)
        return 0

    lax.fori_loop(0, NGRP // 2, body, 0)

    hl = (NGRP - 1) % 2
    for e in range(2):
        @pl.when((NGRP - 1) * GSZ + 6 + e < n)
        def _(e=e):
            drain(hl, 6 + e)
    for e in range(2):
        scat_wait(e, hl, 6 + e)


@functools.partial(
    pl.kernel,
    mesh=_sc_mesh,
    out_type=(
        jax.ShapeDtypeStruct((2, NPAD, 128), _f32),
        jax.ShapeDtypeStruct((2, NPAD), _f32),
    ),
    scratch_types=[
        pltpu.VMEM((2, 8, CH), _i32),
        pltpu.VMEM((2, 8, CH), _i32),
        pltpu.VMEM((2, CH), _i32),
        pltpu.VMEM((2, CH, 128), _f32),
        pltpu.VMEM((2, CH), _f32),
        pltpu.VMEM((64, 128), _f32),
        pltpu.VMEM((ROWS_PER_TILE,), _f32),
        pltpu.VMEM_SHARED((NPAD, 128), _f32),
        pltpu.VMEM_SHARED((NPAD,), _f32),
        pltpu.SemaphoreType.DMA,
        pltpu.SemaphoreType.DMA,
        pltpu.SemaphoreType.DMA,
        pltpu.SemaphoreType.DMA,
        pltpu.SemaphoreType.DMA,
        pltpu.SemaphoreType.DMA,
        pltpu.SemaphoreType.DMA,
        pltpu.SemaphoreType.DMA,
    ],
)
def _prop1_call(p0_hbm, dinv_hbm, src2_hbm, dst2_hbm, s1_hbm, sv_hbm,
                sidx, didx, adjbuf, rows_v, dval_v, zbuf_v, zvec_v,
                acc_sh, sv_sh,
                semi0, semi1, semg0, semg1, semsv0, semsv1, semsc0, semsc1):
    c = lax.axis_index("c")
    s = lax.axis_index("s")
    _zero_vmem_2d(zbuf_v, 64, 128)
    _zero_vmem_1d(zvec_v, ROWS_PER_TILE)
    r0 = s * ROWS_PER_TILE
    for j in range(ROWS_PER_TILE // 64):
        pltpu.sync_copy(zbuf_v, acc_sh.at[pl.ds(r0 + j * 64, 64), :])
    pltpu.sync_copy(zvec_v, sv_sh.at[pl.ds(r0, ROWS_PER_TILE)])
    plsc.subcore_barrier()

    start = s * 80
    n = jnp.minimum(80, NCHUNK - s * 80)
    _edge_pipeline(p0_hbm, src2_hbm, dst2_hbm, dinv_hbm, acc_sh, sv_sh,
                   sidx, didx, adjbuf, rows_v, dval_v,
                   (semi0, semi1), (semg0, semg1), (semsv0, semsv1),
                   (semsc0, semsc1), start, n, c * N, do_sv=True)

    plsc.subcore_barrier()
    sl = pl.ds(r0, ROWS_PER_TILE)
    pltpu.sync_copy(acc_sh.at[sl, :], s1_hbm.at[c, sl, :])
    pltpu.sync_copy(sv_sh.at[sl], sv_hbm.at[c, sl])


@functools.partial(
    pl.kernel,
    mesh=_sc_mesh,
    out_type=jax.ShapeDtypeStruct((4, NPAD, 128), _f32),
    scratch_types=[
        pltpu.VMEM((2, 8, CH), _i32),
        pltpu.VMEM((2, 8, CH), _i32),
        pltpu.VMEM((2, CH), _i32),
        pltpu.VMEM((2, CH, 128), _f32),
        pltpu.VMEM((64, 128), _f32),
        pltpu.VMEM_SHARED((NPAD, 128), _f32),
        pltpu.SemaphoreType.DMA,
        pltpu.SemaphoreType.DMA,
        pltpu.SemaphoreType.DMA,
        pltpu.SemaphoreType.DMA,
        pltpu.SemaphoreType.DMA,
        pltpu.SemaphoreType.DMA,
    ],
)
def _prop2_call(t_hbm, src2_hbm, dst2_hbm, s2_hbm,
                sidx, didx, adjbuf, rows_v, zbuf_v, acc_sh,
                semi0, semi1, semg0, semg1, semsc0, semsc1):
    c = lax.axis_index("c")
    s = lax.axis_index("s")
    _zero_vmem_2d(zbuf_v, 64, 128)
    r0 = s * ROWS_PER_TILE
    start = s * 80
    n = jnp.minimum(80, NCHUNK - s * 80)

    for j in range(2):
        b = c * 2 + j
        for q in range(ROWS_PER_TILE // 64):
            pltpu.sync_copy(zbuf_v, acc_sh.at[pl.ds(r0 + q * 64, 64), :])
        plsc.subcore_barrier()

        _edge_pipeline(t_hbm, src2_hbm, dst2_hbm, None, acc_sh, None,
                       sidx, didx, adjbuf, rows_v, None,
                       (semi0, semi1), (semg0, semg1), None,
                       (semsc0, semsc1), start, n, b * N, do_sv=False)

        plsc.subcore_barrier()
        sl = pl.ds(r0, ROWS_PER_TILE)
        pltpu.sync_copy(acc_sh.at[sl, :], s2_hbm.at[b, sl, :])


def _tc1_body(deg2_ref, x_ref, dinv_ref, p0_ref):
    deg = deg2_ref[0] + deg2_ref[1]
    dinv = lax.rsqrt(jnp.clip(deg, 1.0, None))
    dinv_ref[...] = dinv
    p0 = x_ref[...] * dinv
    p0_ref[0] = p0[:, :128]
    p0_ref[1] = p0[:, 128:]


def _tc1_call(deg2, x):
    return pl.pallas_call(
        _tc1_body,
        grid=(GRID,),
        in_specs=[
            pl.BlockSpec((2, MBLK, 1), lambda i: (0, i, 0)),
            pl.BlockSpec((MBLK, IN_DIM), lambda i: (i, 0)),
        ],
        out_specs=[
            pl.BlockSpec((MBLK, 1), lambda i: (i, 0)),
            pl.BlockSpec((2, MBLK, 128), lambda i: (0, i, 0)),
        ],
        out_shape=[
            jax.ShapeDtypeStruct((N, 1), _f32),
            jax.ShapeDtypeStruct((2, N, 128), _f32),
        ],
    )(deg2, x)


def _tc2_body(s1_ref, sv2_ref, dinv_ref, temb_ref, w1a_ref, w1b_ref, b1_ref,
              w2_ref, t_ref, s_ref):
    dinv = dinv_ref[...]
    sv = (sv2_ref[0] + sv2_ref[1]) * 0.5
    sg = dinv * sv
    s_ref[...] = sg
    x = jnp.concatenate([s1_ref[0], s1_ref[1]], axis=1) * dinv
    v1b = jnp.dot(temb_ref[...], w1b_ref[...],
                  preferred_element_type=_f32) + b1_ref[...]
    h1 = jnp.dot(x, w1a_ref[...], preferred_element_type=_f32) + sg * v1b
    h1 = jnp.maximum(h1, 0.0)
    tt = jnp.dot(h1 * dinv, w2_ref[...], preferred_element_type=_f32)
    for b in range(4):
        t_ref[b] = tt[:, b * 128:(b + 1) * 128]


def _tc2_call(s1, sv2, dinv, temb, w1a, w1b, b1, w2):
    return pl.pallas_call(
        _tc2_body,
        grid=(GRID,),
        in_specs=[
            pl.BlockSpec((2, MBLK, 128), lambda i: (0, i, 0)),
            pl.BlockSpec((2, MBLK, 1), lambda i: (0, i, 0)),
            pl.BlockSpec((MBLK, 1), lambda i: (i, 0)),
            pl.BlockSpec((1, T_DIM), lambda i: (0, 0)),
            pl.BlockSpec((IN_DIM, HID), lambda i: (0, 0)),
            pl.BlockSpec((T_DIM, HID), lambda i: (0, 0)),
            pl.BlockSpec((1, HID), lambda i: (0, 0)),
            pl.BlockSpec((HID, HID), lambda i: (0, 0)),
        ],
        out_specs=[
            pl.BlockSpec((4, MBLK, 128), lambda i: (0, i, 0)),
            pl.BlockSpec((MBLK, 1), lambda i: (i, 0)),
        ],
        out_shape=[
            jax.ShapeDtypeStruct((4, N, 128), _f32),
            jax.ShapeDtypeStruct((N, 1), _f32),
        ],
    )(s1, sv2, dinv, temb, w1a, w1b, b1, w2)


def _tc3_body(s2_ref, dinv_ref, s_ref, b2_ref, wm1_ref, bm1_ref, wm2_ref,
              bm2_ref, out_ref):
    dinv = dinv_ref[...]
    h2 = jnp.concatenate([s2_ref[0], s2_ref[1], s2_ref[2], s2_ref[3]],
                         axis=1) * dinv + s_ref[...] * b2_ref[...]
    z = jnp.maximum(jnp.dot(h2, wm1_ref[...], preferred_element_type=_f32)
                    + bm1_ref[...], 0.0)
    out_ref[...] = jnp.dot(z, wm2_ref[...],
                           preferred_element_type=_f32) + bm2_ref[...]


def _tc3_call(s2, dinv, s, b2, wm1, bm1, wm2, bm2):
    return pl.pallas_call(
        _tc3_body,
        grid=(GRID,),
        in_specs=[
            pl.BlockSpec((4, MBLK, 128), lambda i: (0, i, 0)),
            pl.BlockSpec((MBLK, 1), lambda i: (i, 0)),
            pl.BlockSpec((MBLK, 1), lambda i: (i, 0)),
            pl.BlockSpec((1, HID), lambda i: (0, 0)),
            pl.BlockSpec((HID, HID), lambda i: (0, 0)),
            pl.BlockSpec((1, HID), lambda i: (0, 0)),
            pl.BlockSpec((HID, IN_DIM), lambda i: (0, 0)),
            pl.BlockSpec((1, IN_DIM), lambda i: (0, 0)),
        ],
        out_specs=pl.BlockSpec((MBLK, IN_DIM), lambda i: (i, 0)),
        out_shape=jax.ShapeDtypeStruct((N, IN_DIM), _f32),
    )(s2, dinv, s, b2, wm1, bm1, wm2, bm2)


def kernel(noised_data, t, edge_index, W1, b1, W2, b2, Wm1, bm1, Wm2, bm2):
    x = noised_data[0]
    dst = edge_index[1]
    ei_pad = jnp.pad(edge_index, ((0, 0), (0, CPAD * CH - E)))
    src2 = ei_pad[0].reshape(CPAD, CH)
    dst2 = ei_pad[1].reshape(CPAD, CH)

    half = T_DIM // 2
    freq = jnp.exp(jnp.arange(half, dtype=_f32) * (-math.log(10000.0) / (half - 1)))
    ang = t[0] * freq
    temb = jnp.concatenate([jnp.sin(ang), jnp.cos(ang)])[None]

    deg2 = _deg_call(dst)
    dinv, p0 = _tc1_call(deg2.reshape(2, NPAD, 1), x)
    s1, sv2 = _prop1_call(p0.reshape(2 * N, 128),
                          dinv.reshape(N), src2, dst2)
    tmat, s = _tc2_call(s1, sv2.reshape(2, NPAD, 1), dinv, temb,
                        W1[:IN_DIM], W1[IN_DIM:], b1[None], W2)
    s2 = _prop2_call(tmat.reshape(4 * N, 128), src2, dst2)
    out = _tc3_call(s2, dinv, s, b2[None], Wm1, bm1[None],
                    Wm2, bm2[None])
    return out[None]


# speedup vs baseline: 14.2404x; 1.0593x over previous
import functools
import math

import jax
import jax.numpy as jnp
from jax import lax
from jax.experimental import pallas as pl
from jax.experimental.pallas import tpu as pltpu
from jax.experimental.pallas import tpu_sc as plsc

N = 10000
E = 160000
IN_DIM = 256
T_DIM = 128
HID = 512

NPAD = 10240
CH = 128
NCHUNK = E // CH
NGRP = 10
CPAD = 1280
ROWS_PER_TILE = NPAD // 16
MBLK = 1000
GRID = N // MBLK

_f32 = jnp.float32
_i32 = jnp.int32


def _zero_vmem_1d(ref, nwords):
    def body(i, _):
        ref[pl.ds(i * 16, 16)] = jnp.zeros((16,), _f32)
        return 0
    lax.fori_loop(0, nwords // 16, body, 0)


def _zero_vmem_2d(ref, rows, cols):
    per_row = cols // 16
    def body(i, _):
        r = i // per_row
        c = (i % per_row) * 16
        ref[r, pl.ds(c, 16)] = jnp.zeros((16,), _f32)
        return 0
    lax.fori_loop(0, rows * per_row, body, 0)


_sc_mesh = plsc.VectorSubcoreMesh(core_axis_name="c", subcore_axis_name="s")


@functools.partial(
    pl.kernel,
    mesh=_sc_mesh,
    out_type=jax.ShapeDtypeStruct((2, NPAD), _f32),
    scratch_types=[
        pltpu.VMEM((2, 8, CH), _i32),
        pltpu.VMEM((CH,), _f32),
        pltpu.VMEM((ROWS_PER_TILE,), _f32),
        pltpu.VMEM_SHARED((NPAD,), _f32),
        pltpu.SemaphoreType.DMA,
        pltpu.SemaphoreType.DMA,
    ],
)
def _deg_call(dst2_hbm, out_hbm, didx, ones_v, zvec_v, acc_sh, semi0, semi1):
    c = lax.axis_index("c")
    s = lax.axis_index("s")
    semi = (semi0, semi1)
    _zero_vmem_1d(zvec_v, ROWS_PER_TILE)
    def fill1(i, _):
        ones_v[pl.ds(i * 16, 16)] = jnp.ones((16,), _f32)
        return 0
    lax.fori_loop(0, CH // 16, fill1, 0)
    pltpu.sync_copy(zvec_v, acc_sh.at[pl.ds(s * ROWS_PER_TILE, ROWS_PER_TILE)])
    plsc.subcore_barrier()

    start = c * 640 + s * 40
    n = jnp.where(c == 0, 40, jnp.minimum(40, 610 - s * 40))

    def idx_issue(slot, g):
        pltpu.async_copy(dst2_hbm.at[pl.ds(start + g * 8, 8), :],
                         didx.at[slot], semi[slot])

    def idx_wait(slot, g):
        pltpu.make_async_copy(dst2_hbm.at[pl.ds(start + g * 8, 8), :],
                              didx.at[slot], semi[slot]).wait()

    idx_issue(0, 0)
    for g in range(5):
        idx_wait(g % 2, g)
        if g + 1 < 5:
            idx_issue((g + 1) % 2, g + 1)
        for u in range(8):
            @pl.when(g * 8 + u < n)
            def _(g=g, u=u):
                pltpu.sync_copy(ones_v, acc_sh.at[didx.at[g % 2, u]],
                                add=True)

    plsc.subcore_barrier()
    sl = pl.ds(s * ROWS_PER_TILE, ROWS_PER_TILE)
    pltpu.sync_copy(acc_sh.at[sl], out_hbm.at[c, sl])


def _edge_pipeline(tbl_hbm, src2_hbm, dst2_hbm, dinv_hbm, acc_sh, sv_sh,
                   sidx, didx, adjbuf, rows_v, dval_v, semi, semg, semsv,
                   semsc, start, n, off, do_sv, sv_sel=None):
    GSZ = 8

    def idx_issue(slot, g):
        g8 = start + g * GSZ
        pltpu.async_copy(src2_hbm.at[pl.ds(g8, GSZ), :], sidx.at[slot],
                         semi[slot])
        pltpu.async_copy(dst2_hbm.at[pl.ds(g8, GSZ), :], didx.at[slot],
                         semi[slot])

    def idx_wait(slot, g):
        g8 = start + g * GSZ
        pltpu.make_async_copy(src2_hbm.at[pl.ds(g8, GSZ), :], sidx.at[slot],
                              semi[slot]).wait()
        pltpu.make_async_copy(dst2_hbm.at[pl.ds(g8, GSZ), :], didx.at[slot],
                              semi[slot]).wait()

    def scat_wait(w, slot, u):
        pltpu.make_async_copy(rows_v.at[w], acc_sh.at[didx.at[slot, u]],
                              semsc[w]).wait()

    def fire(slot, u, first_group):
        w = u % 2
        if u >= 2:
            scat_wait(w, slot, u - 2)
        else:
            @pl.when(jnp.logical_not(first_group))
            def _():
                scat_wait(w, 1 - slot, u + 6)
        if do_sv:
            @pl.when(u % 2 == sv_sel)
            def _():
                pltpu.async_copy(dinv_hbm.at[sidx.at[slot, u]], dval_v.at[w],
                                 semsv[w])
        for k in range(CH // 16):
            adjbuf[w, pl.ds(k * 16, 16)] = sidx[slot, u, pl.ds(k * 16, 16)] + off
        pltpu.async_copy(tbl_hbm.at[adjbuf.at[w]], rows_v.at[w], semg[w])

    def drain(slot, u):
        w = u % 2
        pltpu.make_async_copy(tbl_hbm.at[adjbuf.at[w]], rows_v.at[w],
                              semg[w]).wait()
        pltpu.async_copy(rows_v.at[w], acc_sh.at[didx.at[slot, u]], semsc[w],
                         add=True)
        if do_sv:
            @pl.when(u % 2 == sv_sel)
            def _():
                pltpu.make_async_copy(dinv_hbm.at[sidx.at[slot, u]],
                                      dval_v.at[w], semsv[w]).wait()
                pltpu.sync_copy(dval_v.at[w], sv_sh.at[didx.at[slot, u]],
                                add=True)

    idx_issue(0, 0)

    def body(gg, _):
        for h in (0, 1):
            g = gg * 2 + h
            idx_wait(h, g)
            for u in range(GSZ):
                q_drain = g * GSZ + u - 2
                slot_d, u_d = ((1 - h, u + 6) if u < 2 else (h, u - 2))

                @pl.when(jnp.logical_and(q_drain >= 0, q_drain < n))
                def _(slot_d=slot_d, u_d=u_d):
                    drain(slot_d, u_d)

                @pl.when(g * GSZ + u < n)
                def _(h=h, u=u, gg_=gg):
                    fire(h, u, jnp.logical_and(gg_ == 0, h == 0))

                if u == 4:
                    @pl.when(g + 1 < NGRP)
                    def _(h=h, g=g):
                        idx_issue(1 - h, g + 1)
        return 0

    lax.fori_loop(0, NGRP // 2, body, 0)

    hl = (NGRP - 1) % 2
    for e in range(2):
        @pl.when((NGRP - 1) * GSZ + 6 + e < n)
        def _(e=e):
            drain(hl, 6 + e)
    for e in range(2):
        scat_wait(e, hl, 6 + e)


@functools.partial(
    pl.kernel,
    mesh=_sc_mesh,
    out_type=(
        jax.ShapeDtypeStruct((2, NPAD, 128), _f32),
        jax.ShapeDtypeStruct((2, NPAD), _f32),
    ),
    scratch_types=[
        pltpu.VMEM((2, 8, CH), _i32),
        pltpu.VMEM((2, 8, CH), _i32),
        pltpu.VMEM((2, CH), _i32),
        pltpu.VMEM((2, CH, 128), _f32),
        pltpu.VMEM((2, CH), _f32),
        pltpu.VMEM((64, 128), _f32),
        pltpu.VMEM((ROWS_PER_TILE,), _f32),
        pltpu.VMEM_SHARED((NPAD, 128), _f32),
        pltpu.VMEM_SHARED((NPAD,), _f32),
        pltpu.SemaphoreType.DMA,
        pltpu.SemaphoreType.DMA,
        pltpu.SemaphoreType.DMA,
        pltpu.SemaphoreType.DMA,
        pltpu.SemaphoreType.DMA,
        pltpu.SemaphoreType.DMA,
        pltpu.SemaphoreType.DMA,
        pltpu.SemaphoreType.DMA,
    ],
)
def _prop1_call(p0_hbm, dinv_hbm, src2_hbm, dst2_hbm, s1_hbm, sv_hbm,
                sidx, didx, adjbuf, rows_v, dval_v, zbuf_v, zvec_v,
                acc_sh, sv_sh,
                semi0, semi1, semg0, semg1, semsv0, semsv1, semsc0, semsc1):
    c = lax.axis_index("c")
    s = lax.axis_index("s")
    _zero_vmem_2d(zbuf_v, 64, 128)
    _zero_vmem_1d(zvec_v, ROWS_PER_TILE)
    r0 = s * ROWS_PER_TILE
    for j in range(ROWS_PER_TILE // 64):
        pltpu.sync_copy(zbuf_v, acc_sh.at[pl.ds(r0 + j * 64, 64), :])
    pltpu.sync_copy(zvec_v, sv_sh.at[pl.ds(r0, ROWS_PER_TILE)])
    plsc.subcore_barrier()

    start = s * 80
    n = jnp.minimum(80, NCHUNK - s * 80)
    _edge_pipeline(p0_hbm, src2_hbm, dst2_hbm, dinv_hbm, acc_sh, sv_sh,
                   sidx, didx, adjbuf, rows_v, dval_v,
                   (semi0, semi1), (semg0, semg1), (semsv0, semsv1),
                   (semsc0, semsc1), start, n, c * N, do_sv=True, sv_sel=c)

    plsc.subcore_barrier()
    sl = pl.ds(r0, ROWS_PER_TILE)
    pltpu.sync_copy(acc_sh.at[sl, :], s1_hbm.at[c, sl, :])
    pltpu.sync_copy(sv_sh.at[sl], sv_hbm.at[c, sl])


@functools.partial(
    pl.kernel,
    mesh=_sc_mesh,
    out_type=jax.ShapeDtypeStruct((4, NPAD, 128), _f32),
    scratch_types=[
        pltpu.VMEM((2, 8, CH), _i32),
        pltpu.VMEM((2, 8, CH), _i32),
        pltpu.VMEM((2, CH), _i32),
        pltpu.VMEM((2, CH, 128), _f32),
        pltpu.VMEM((64, 128), _f32),
        pltpu.VMEM_SHARED((NPAD, 128), _f32),
        pltpu.SemaphoreType.DMA,
        pltpu.SemaphoreType.DMA,
        pltpu.SemaphoreType.DMA,
        pltpu.SemaphoreType.DMA,
        pltpu.SemaphoreType.DMA,
        pltpu.SemaphoreType.DMA,
    ],
)
def _prop2_call(t_hbm, src2_hbm, dst2_hbm, s2_hbm,
                sidx, didx, adjbuf, rows_v, zbuf_v, acc_sh,
                semi0, semi1, semg0, semg1, semsc0, semsc1):
    c = lax.axis_index("c")
    s = lax.axis_index("s")
    _zero_vmem_2d(zbuf_v, 64, 128)
    r0 = s * ROWS_PER_TILE
    start = s * 80
    n = jnp.minimum(80, NCHUNK - s * 80)

    for j in range(2):
        b = c * 2 + j
        for q in range(ROWS_PER_TILE // 64):
            pltpu.sync_copy(zbuf_v, acc_sh.at[pl.ds(r0 + q * 64, 64), :])
        plsc.subcore_barrier()

        _edge_pipeline(t_hbm, src2_hbm, dst2_hbm, None, acc_sh, None,
                       sidx, didx, adjbuf, rows_v, None,
                       (semi0, semi1), (semg0, semg1), None,
                       (semsc0, semsc1), start, n, b * N, do_sv=False)

        plsc.subcore_barrier()
        sl = pl.ds(r0, ROWS_PER_TILE)
        pltpu.sync_copy(acc_sh.at[sl, :], s2_hbm.at[b, sl, :])


def _tc1_body(deg2_ref, x_ref, dinv_ref, p0_ref):
    deg = deg2_ref[0] + deg2_ref[1]
    dinv = lax.rsqrt(jnp.clip(deg, 1.0, None))
    dinv_ref[...] = dinv
    p0 = x_ref[...] * dinv
    p0_ref[0] = p0[:, :128]
    p0_ref[1] = p0[:, 128:]


def _tc1_call(deg2, x):
    return pl.pallas_call(
        _tc1_body,
        grid=(GRID,),
        in_specs=[
            pl.BlockSpec((2, MBLK, 1), lambda i: (0, i, 0)),
            pl.BlockSpec((MBLK, IN_DIM), lambda i: (i, 0)),
        ],
        out_specs=[
            pl.BlockSpec((MBLK, 1), lambda i: (i, 0)),
            pl.BlockSpec((2, MBLK, 128), lambda i: (0, i, 0)),
        ],
        out_shape=[
            jax.ShapeDtypeStruct((N, 1), _f32),
            jax.ShapeDtypeStruct((2, N, 128), _f32),
        ],
    )(deg2, x)


def _tc2_body(s1_ref, sv2_ref, dinv_ref, temb_ref, w1a_ref, w1b_ref, b1_ref,
              w2_ref, t_ref, s_ref):
    dinv = dinv_ref[...]
    sv = sv2_ref[0] + sv2_ref[1]
    sg = dinv * sv
    s_ref[...] = sg
    x = jnp.concatenate([s1_ref[0], s1_ref[1]], axis=1) * dinv
    v1b = jnp.dot(temb_ref[...], w1b_ref[...],
                  preferred_element_type=_f32) + b1_ref[...]
    h1 = jnp.dot(x, w1a_ref[...], preferred_element_type=_f32) + sg * v1b
    h1 = jnp.maximum(h1, 0.0)
    tt = jnp.dot(h1 * dinv, w2_ref[...], preferred_element_type=_f32)
    for b in range(4):
        t_ref[b] = tt[:, b * 128:(b + 1) * 128]


def _tc2_call(s1, sv2, dinv, temb, w1a, w1b, b1, w2):
    return pl.pallas_call(
        _tc2_body,
        grid=(GRID,),
        in_specs=[
            pl.BlockSpec((2, MBLK, 128), lambda i: (0, i, 0)),
            pl.BlockSpec((2, MBLK, 1), lambda i: (0, i, 0)),
            pl.BlockSpec((MBLK, 1), lambda i: (i, 0)),
            pl.BlockSpec((1, T_DIM), lambda i: (0, 0)),
            pl.BlockSpec((IN_DIM, HID), lambda i: (0, 0)),
            pl.BlockSpec((T_DIM, HID), lambda i: (0, 0)),
            pl.BlockSpec((1, HID), lambda i: (0, 0)),
            pl.BlockSpec((HID, HID), lambda i: (0, 0)),
        ],
        out_specs=[
            pl.BlockSpec((4, MBLK, 128), lambda i: (0, i, 0)),
            pl.BlockSpec((MBLK, 1), lambda i: (i, 0)),
        ],
        out_shape=[
            jax.ShapeDtypeStruct((4, N, 128), _f32),
            jax.ShapeDtypeStruct((N, 1), _f32),
        ],
    )(s1, sv2, dinv, temb, w1a, w1b, b1, w2)


def _tc3_body(s2_ref, dinv_ref, s_ref, b2_ref, wm1_ref, bm1_ref, wm2_ref,
              bm2_ref, out_ref):
    dinv = dinv_ref[...]
    h2 = jnp.concatenate([s2_ref[0], s2_ref[1], s2_ref[2], s2_ref[3]],
                         axis=1) * dinv + s_ref[...] * b2_ref[...]
    z = jnp.maximum(jnp.dot(h2, wm1_ref[...], preferred_element_type=_f32)
                    + bm1_ref[...], 0.0)
    out_ref[...] = jnp.dot(z, wm2_ref[...],
                           preferred_element_type=_f32) + bm2_ref[...]


def _tc3_call(s2, dinv, s, b2, wm1, bm1, wm2, bm2):
    return pl.pallas_call(
        _tc3_body,
        grid=(GRID,),
        in_specs=[
            pl.BlockSpec((4, MBLK, 128), lambda i: (0, i, 0)),
            pl.BlockSpec((MBLK, 1), lambda i: (i, 0)),
            pl.BlockSpec((MBLK, 1), lambda i: (i, 0)),
            pl.BlockSpec((1, HID), lambda i: (0, 0)),
            pl.BlockSpec((HID, HID), lambda i: (0, 0)),
            pl.BlockSpec((1, HID), lambda i: (0, 0)),
            pl.BlockSpec((HID, IN_DIM), lambda i: (0, 0)),
            pl.BlockSpec((1, IN_DIM), lambda i: (0, 0)),
        ],
        out_specs=pl.BlockSpec((MBLK, IN_DIM), lambda i: (i, 0)),
        out_shape=jax.ShapeDtypeStruct((N, IN_DIM), _f32),
    )(s2, dinv, s, b2, wm1, bm1, wm2, bm2)


def kernel(noised_data, t, edge_index, W1, b1, W2, b2, Wm1, bm1, Wm2, bm2):
    x = noised_data[0]
    ei_pad = jnp.pad(edge_index, ((0, 0), (0, CPAD * CH - E)))
    src2 = ei_pad[0].reshape(CPAD, CH)
    dst2 = ei_pad[1].reshape(CPAD, CH)

    half = T_DIM // 2
    freq = jnp.exp(jnp.arange(half, dtype=_f32) * (-math.log(10000.0) / (half - 1)))
    ang = t[0] * freq
    temb = jnp.concatenate([jnp.sin(ang), jnp.cos(ang)])[None]

    deg2 = _deg_call(dst2)
    dinv, p0 = _tc1_call(deg2.reshape(2, NPAD, 1), x)
    s1, sv2 = _prop1_call(p0.reshape(2 * N, 128),
                          dinv.reshape(N), src2, dst2)
    tmat, s = _tc2_call(s1, sv2.reshape(2, NPAD, 1), dinv, temb,
                        W1[:IN_DIM], W1[IN_DIM:], b1[None], W2)
    s2 = _prop2_call(tmat.reshape(4 * N, 128), src2, dst2)
    out = _tc3_call(s2, dinv, s, b2[None], Wm1, bm1[None],
                    Wm2, bm2[None])
    return out[None]
